# baseline (speedup 1.0000x reference)
import functools

import jax
import jax.numpy as jnp
from jax import lax
from jax.experimental import pallas as pl
from jax.experimental.pallas import tpu as pltpu
from jax.experimental.pallas import tpu_sc as plsc

_SAMPLE_TO = 1000
_LANES = 16
_IDX_PAD = 1008


def _build_gather(rows: int, k: int):
  info = plsc.get_sparse_core_info()
  nw = info.num_cores * info.num_subcores
  rows_per_w = rows // nw
  n_chunks = _IDX_PAD // _LANES

  mesh = plsc.VectorSubcoreMesh(core_axis_name="c", subcore_axis_name="s")

  @functools.partial(
      pl.kernel,
      mesh=mesh,
      out_type=jax.ShapeDtypeStruct((rows * _SAMPLE_TO,), jnp.float32),
      compiler_params=pltpu.CompilerParams(needs_layout_passes=False),
      scratch_types=[
          pltpu.VMEM((k,), jnp.float32),
          pltpu.VMEM((_IDX_PAD,), jnp.int32),
          pltpu.VMEM((_IDX_PAD,), jnp.float32),
      ],
  )
  def gather_kernel(flat_hbm, idx_hbm, out_hbm, row_v, idx_v, out_v):
    wid = lax.axis_index("s") * info.num_cores + lax.axis_index("c")
    pltpu.sync_copy(idx_hbm, idx_v)

    def per_row(i, _):
      r = wid * rows_per_w + i
      src = pl.multiple_of(r * k, 8)
      pltpu.sync_copy(flat_hbm.at[pl.ds(src, k)], row_v)
      for j in range(n_chunks):
        iv = idx_v[pl.ds(j * _LANES, _LANES)]
        out_v[pl.ds(j * _LANES, _LANES)] = plsc.load_gather(row_v, [iv])
      dst = pl.multiple_of(r * _SAMPLE_TO, 8)
      pltpu.sync_copy(out_v.at[pl.ds(0, _SAMPLE_TO)],
                      out_hbm.at[pl.ds(dst, _SAMPLE_TO)])
      return _

    lax.fori_loop(0, rows_per_w, per_row, None)

  return gather_kernel


def kernel(inputs):
  rows, k = inputs.shape
  if k <= _SAMPLE_TO:
    return inputs
  perm = jax.random.permutation(jax.random.key(42), k)
  ridxs = perm[:_SAMPLE_TO].astype(jnp.int32)
  ridxs = jnp.concatenate(
      [ridxs, jnp.full((_IDX_PAD - _SAMPLE_TO,), ridxs[-1], jnp.int32)])
  flat = inputs.reshape(-1)
  out = _build_gather(rows, k)(flat, ridxs)
  return out.reshape(rows, _SAMPLE_TO)


# speedup vs baseline: 1.9417x; 1.9417x over previous
import functools

import jax
import jax.numpy as jnp
import numpy as np
from jax import lax
from jax.experimental import pallas as pl
from jax.experimental.pallas import tpu as pltpu
from jax.experimental.pallas import tpu_sc as plsc

_SAMPLE_TO = 1000
_LANES = 16
_CHUNK_COLS = 12544
_OUT_PAD = 1008


@functools.lru_cache(maxsize=None)
def _plan(k: int):
  with jax.ensure_compile_time_eval():
    ridxs = np.asarray(jax.random.permutation(jax.random.key(42), k))
  ridxs = ridxs[:_SAMPLE_TO].astype(np.int64)
  cover = -(-(int(ridxs.max()) + 1) // 128) * 128
  n_chunks = -(-cover // _CHUNK_COLS)
  last_w = cover - (n_chunks - 1) * _CHUNK_COLS
  cols, pos = [], []
  for c in range(n_chunks):
    lo = c * _CHUNK_COLS
    hi = min(lo + _CHUNK_COLS, cover)
    sel = np.where((ridxs >= lo) & (ridxs < hi))[0]
    cols.append(ridxs[sel] - lo)
    pos.append(sel)
  p = max(len(x) for x in cols)
  p = -(-p // _LANES) * _LANES
  cols_arr = np.zeros((n_chunks, p), np.int32)
  pos_arr = np.zeros((n_chunks, p), np.int32)
  for c in range(n_chunks):
    cols_arr[c] = cols[c][0]
    pos_arr[c] = pos[c][0]
    cols_arr[c, : len(cols[c])] = cols[c]
    pos_arr[c, : len(pos[c])] = pos[c]
  return cols_arr, pos_arr, n_chunks, p, last_w


def _build_kernel(rows: int, k: int, n_chunks: int, p: int, last_w: int):
  info = plsc.get_sparse_core_info()
  nw = info.num_cores * info.num_subcores
  n_groups = rows // 8
  gpw = n_groups // nw

  mesh = plsc.VectorSubcoreMesh(core_axis_name="c", subcore_axis_name="s")

  @functools.partial(
      pl.kernel,
      mesh=mesh,
      out_type=jax.ShapeDtypeStruct((rows, _SAMPLE_TO), jnp.float32),
      compiler_params=pltpu.CompilerParams(needs_layout_passes=False),
      scratch_types=[
          pltpu.VMEM((8, _CHUNK_COLS), jnp.float32),
          pltpu.VMEM((8, _SAMPLE_TO), jnp.float32),
          pltpu.VMEM((n_chunks, p), jnp.int32),
          pltpu.VMEM((n_chunks, p), jnp.int32),
      ],
  )
  def ds_kernel(in_hbm, cols_hbm, pos_hbm, out_hbm, chunk_v, out_v,
                cols_v, pos_v):
    wid = lax.axis_index("s") * info.num_cores + lax.axis_index("c")
    pltpu.sync_copy(cols_hbm, cols_v)
    pltpu.sync_copy(pos_hbm, pos_v)

    def per_group(gi, _):
      g8 = pl.multiple_of((wid * gpw + gi) * 8, 8)
      for c in range(n_chunks):
        w = _CHUNK_COLS if c < n_chunks - 1 else last_w
        pltpu.sync_copy(in_hbm.at[pl.ds(g8, 8), pl.ds(c * _CHUNK_COLS, w)],
                        chunk_v.at[:, pl.ds(0, w)])
        for t in range(p // _LANES):
          cv = cols_v[c, pl.ds(t * _LANES, _LANES)]
          pv = pos_v[c, pl.ds(t * _LANES, _LANES)]
          for r in range(8):
            rv = jnp.full((_LANES,), r, jnp.int32)
            vals = plsc.load_gather(chunk_v, [rv, cv])
            plsc.store_scatter(out_v, [rv, pv], vals)
      pltpu.sync_copy(out_v, out_hbm.at[pl.ds(g8, 8)])
      return _

    lax.fori_loop(0, gpw, per_group, None)

  return ds_kernel


def kernel(inputs):
  rows, k = inputs.shape
  if k <= _SAMPLE_TO:
    return inputs
  cols_arr, pos_arr, n_chunks, p, last_w = _plan(k)
  fn = _build_kernel(rows, k, n_chunks, p, last_w)
  return fn(inputs, jnp.asarray(cols_arr), jnp.asarray(pos_arr))
